# baseline (speedup 1.0000x reference)
import math

import numpy as np
import jax
import jax.numpy as jnp
from jax import lax
from jax.experimental import pallas as pl
from jax.experimental.pallas import tpu as pltpu
from jax.experimental.pallas import tpu_sc as plsc

_NUM_EMB = 1000000
_DIM = 64
_BATCH = 4096
_SEQ = 200
_LANES = 16

_NC, _NS = 2, 16
_NW = _NC * _NS
_BBLK = _BATCH // _NW
_NG = _BBLK // _LANES


def _pos_encoding():
    pos = np.arange(_SEQ, dtype=np.float32)[:, None]
    div = np.exp(np.arange(0, _DIM, 2, dtype=np.float32)
                 * -(math.log(10000.0) / _DIM))
    pe = np.zeros((_SEQ, _DIM), dtype=np.float32)
    pe[:, 0::2] = np.sin(pos * div)
    pe[:, 1::2] = np.cos(pos * div)
    return pe.reshape(-1)


_PE = _pos_encoding()


def _body(xt_hbm, pe_hbm, table_hbm, out_hbm,
          xv_v, idxp_v, gath_v, ostage_v, pe_v, sem):
    wid = lax.axis_index("s") * _NC + lax.axis_index("c")
    b0 = wid * _BBLK
    pltpu.sync_copy(pe_hbm, pe_v)
    iota = lax.iota(jnp.int32, _LANES)
    rowids = [g * _LANES + iota for g in range(_NG)]

    def step(l, carry):
        pltpu.sync_copy(xt_hbm.at[l, pl.ds(b0, _BBLK)], xv_v)
        for g in range(_NG):
            sl = pl.ds(g * _LANES, _LANES)
            idxp_v[sl] = xv_v[sl] >> 1
        pltpu.async_copy(table_hbm.at[idxp_v], gath_v, sem).wait()

        pars = []
        for g in range(_NG):
            xv = xv_v[pl.ds(g * _LANES, _LANES)]
            pars.append((xv & 1) << 6)
        pe_base = iota * 0 + l * _DIM

        def col(d, c):
            pv = plsc.load_gather(pe_v, [pe_base + d])
            for g in range(_NG):
                cv = plsc.load_gather(gath_v, [rowids[g], pars[g] + d])
                ostage_v[d, pl.ds(g * _LANES, _LANES)] = cv + pv
            return c

        lax.fori_loop(0, _DIM, col, 0)
        pltpu.sync_copy(ostage_v, out_hbm.at[l, :, pl.ds(b0, _BBLK)])
        return carry

    lax.fori_loop(0, _SEQ, step, 0)


@jax.jit
def _run(xt, pe, table2):
    mesh = plsc.VectorSubcoreMesh(core_axis_name="c", subcore_axis_name="s")
    f = pl.kernel(
        _body,
        out_type=jax.ShapeDtypeStruct((_SEQ, _DIM, _BATCH), jnp.float32),
        mesh=mesh,
        scratch_types=[
            pltpu.VMEM((_BBLK,), jnp.int32),
            pltpu.VMEM((_BBLK,), jnp.int32),
            pltpu.VMEM((_BBLK, 2 * _DIM), jnp.float32),
            pltpu.VMEM((_DIM, _BBLK), jnp.float32),
            pltpu.VMEM((_SEQ * _DIM,), jnp.float32),
            pltpu.SemaphoreType.DMA,
        ],
        compiler_params=pltpu.CompilerParams(needs_layout_passes=False),
    )
    return f(xt, pe, table2)


def kernel(x, table):
    table2 = table.reshape(_NUM_EMB // 2, 2 * _DIM)
    out2 = _run(x.T, _PE, table2)
    return jnp.transpose(out2, (2, 0, 1))
